# baseline (speedup 1.0000x reference)
import functools

import numpy as np
import jax
import jax.numpy as jnp
from jax import lax
from jax.experimental import pallas as pl
from jax.experimental.pallas import tpu as pltpu
from jax.experimental.pallas import tpu_sc as plsc

_B, _SLEN, _D = 8, 200, 300
_MAXLEN, _POSI, _WINDOW, _NBASES = 200, 32, 10, 4
_RELNUM = _WINDOW + 2
_EMO = 200


def _build_M() -> np.ndarray:
    i = np.arange(_SLEN)[:, None]
    j = np.arange(_SLEN)[None, :]
    d = i - j
    lower = -np.minimum((d + 1) // 2, _WINDOW + 1)
    rel_adj = np.where(j > i, 1, np.where(j == i, 0, lower))
    etype = np.mod(rel_adj, _RELNUM)
    M = np.zeros((_RELNUM, _SLEN, _SLEN), np.float32)
    for r in range(_RELNUM):
        mask = (etype == r).astype(np.float32)
        cnt = mask.sum(axis=0)
        Mr = np.where(cnt[None, :] > 0, mask / np.maximum(cnt[None, :], 1.0), 0.0)
        M[r] = Mr.T
    return M


_M_CONST = _build_M()


_IBLK = 8


def _fused_body(comp_ref, x_ref, pad_ref, M_ref, basis_ref, root_ref,
                bias_ref, Wa_ref, ba_ref, Wb_ref, bb_ref, xk_ref, xv_ref,
                final_ref, outk_ref, outv_ref, n_ref):
    step = pl.program_id(0)

    for x3_ref, out_ref in ((xk_ref, outk_ref), (xv_ref, outv_ref)):
        for t in range(_IBLK):
            w = x3_ref[t]
            lo = w[:_POSI]
            hi = w[_POSI:, :_SLEN - 128]
            full = jnp.concatenate([lo, hi], axis=1)
            for b in range(_B):
                out_ref[b, t] = full

    @pl.when(step == 0)
    def _():
        for b in range(_NBASES):
            Nb = comp_ref[0, b] * M_ref[0]
            for r in range(1, _RELNUM):
                Nb = Nb + comp_ref[r, b] * M_ref[r]
            n_ref[b] = Nb

    @pl.when(step < _B)
    def _():
        x = x_ref[0]
        pad = pad_ref[0]
        acc = jnp.dot(x, root_ref[...],
                      preferred_element_type=jnp.float32) + bias_ref[...]
        for b in range(_NBASES):
            acc = acc + jnp.dot(
                jnp.dot(n_ref[b], x, preferred_element_type=jnp.float32),
                basis_ref[b], preferred_element_type=jnp.float32)
        ma = jnp.dot(acc, Wa_ref[...],
                     preferred_element_type=jnp.float32) + ba_ref[...]
        mb = jnp.dot(pad, Wb_ref[...],
                     preferred_element_type=jnp.float32) + bb_ref[...]
        S = lax.dot_general(ma, mb, (((1,), (1,)), ((), ())),
                            preferred_element_type=jnp.float32)
        colmax = jnp.max(S, axis=0, keepdims=True)
        E = jnp.exp(S - colmax)
        E = E / jnp.sum(E, axis=0, keepdims=True)
        final_ref[0] = jnp.dot(E, pad, preferred_element_type=jnp.float32)


def _fused_call(comp, x, padded, M, basis, root, bias2, Wa, ba2, Wb, bb2,
                x3k, x3v):
    full = lambda shape: pl.BlockSpec(shape, lambda i: (0,) * len(shape))
    bspec = lambda: pl.BlockSpec(
        (1, _SLEN, _D), lambda i: (jnp.minimum(i, _B - 1), 0, 0))
    inspec = pl.BlockSpec((_IBLK, 2 * _POSI, 128), lambda i: (i, 0, 0))
    outspec = pl.BlockSpec((_B, _IBLK, _POSI, _SLEN), lambda i: (0, i, 0, 0))
    oshape4 = jax.ShapeDtypeStruct((_B, _SLEN, _POSI, _SLEN), jnp.float32)
    return pl.pallas_call(
        _fused_body,
        grid=(_SLEN // _IBLK,),
        in_specs=[
            pl.BlockSpec(memory_space=pltpu.SMEM),
            bspec(),
            bspec(),
            full((_RELNUM, _SLEN, _SLEN)),
            full((_NBASES, _D, _D)),
            full((_D, _D)),
            full((1, _D)),
            full((_D, 600)),
            full((1, 600)),
            full((_D, 600)),
            full((1, 600)),
            inspec,
            inspec,
        ],
        out_specs=(pl.BlockSpec((1, _SLEN, _D),
                                lambda i: (jnp.minimum(i, _B - 1), 0, 0)),
                   outspec, outspec),
        out_shape=(jax.ShapeDtypeStruct((_B, _SLEN, _D), jnp.float32),
                   oshape4, oshape4),
        scratch_shapes=[pltpu.VMEM((_NBASES, _SLEN, _SLEN), jnp.float32)],
    )(comp, x, padded, M, basis, root, bias2, Wa, ba2, Wb, bb2, x3k, x3v)


_NROWS = _B * _SLEN
_ROWW = _SLEN * _POSI
_TBL = 400 * _POSI
_NWORK = 32


@functools.cache
def _make_sc_rel_emb():
    @functools.partial(
        pl.kernel,
        out_type=jax.ShapeDtypeStruct((_SLEN * _POSI * 256,), jnp.float32),
        scratch_types=[pltpu.VMEM((_POSI * 512,), jnp.float32),
                       pltpu.VMEM((_POSI * 256,), jnp.float32)],
        mesh=plsc.VectorSubcoreMesh(core_axis_name="c", subcore_axis_name="s"),
        compiler_params=pltpu.CompilerParams(needs_layout_passes=False),
    )
    def _sc_rel_emb(qt_hbm, out_hbm, qt_v, win_v):
        wid = lax.axis_index("s") * 2 + lax.axis_index("c")
        pltpu.sync_copy(qt_hbm, qt_v)
        lanes = lax.iota(jnp.int32, 16)

        def job_body(t, carry):
            i = wid + t * _NWORK

            @pl.when(i < _SLEN)
            def _():
                tbase = 199 - i

                def p_body(p, c):
                    rowbase = tbase + p * 512
                    lo = pl.multiple_of(p * 128, 8)
                    hi = pl.multiple_of(_POSI * 128 + p * 128, 8)
                    for j0 in range(0, 128, 16):
                        vals = plsc.load_gather(qt_v, [rowbase + j0 + lanes])
                        win_v[pl.ds(lo + j0, 16)] = vals
                    for j0 in range(128, 256, 16):
                        vals = plsc.load_gather(qt_v, [rowbase + j0 + lanes])
                        win_v[pl.ds(hi + (j0 - 128), 16)] = vals
                    return c

                lax.fori_loop(0, _POSI, p_body, 0)

                dst = pl.multiple_of(i * (_POSI * 256), 8)
                pltpu.sync_copy(win_v.at[pl.ds(0, _POSI * 256)],
                                out_hbm.at[pl.ds(dst, _POSI * 256)])

            return carry

        lax.fori_loop(0, 7, job_body, 0)

    return _sc_rel_emb


def kernel(x, adj_index, emo_emb, pe_k, pe_v, comp, basis, root, bias,
           Wa, ba, Wb, bb):
    del adj_index
    padded = jnp.concatenate(
        [emo_emb, jnp.zeros((_B, _SLEN, _D - _EMO), jnp.float32)], axis=-1)
    M = jnp.asarray(_M_CONST)
    ztail = jnp.zeros((_POSI, 512 - (_MAXLEN + 1)), jnp.float32)
    qtk = jnp.concatenate([pe_k[::-1].T, ztail], axis=1).reshape(-1)
    qtv = jnp.concatenate([pe_v[::-1].T, ztail], axis=1).reshape(-1)
    sc = _make_sc_rel_emb()
    outk = sc(qtk).reshape(_SLEN, 2 * _POSI, 128)
    outv = sc(qtv).reshape(_SLEN, 2 * _POSI, 128)
    final, bk, bv = _fused_call(
        comp, x, padded, M, basis, root, bias.reshape(1, _D), Wa,
        ba.reshape(1, 600), Wb, bb.reshape(1, 600), outk, outv)
    rel_emb_k = jnp.swapaxes(bk, 2, 3)
    rel_emb_v = jnp.swapaxes(bv, 2, 3)
    return final, rel_emb_k, rel_emb_v


# speedup vs baseline: 1.1392x; 1.1392x over previous
import functools

import numpy as np
import jax
import jax.numpy as jnp
from jax import lax
from jax.experimental import pallas as pl
from jax.experimental.pallas import tpu as pltpu
from jax.experimental.pallas import tpu_sc as plsc

_B, _SLEN, _D = 8, 200, 300
_MAXLEN, _POSI, _WINDOW, _NBASES = 200, 32, 10, 4
_RELNUM = _WINDOW + 2
_EMO = 200


def _build_M() -> np.ndarray:
    i = np.arange(_SLEN)[:, None]
    j = np.arange(_SLEN)[None, :]
    d = i - j
    lower = -np.minimum((d + 1) // 2, _WINDOW + 1)
    rel_adj = np.where(j > i, 1, np.where(j == i, 0, lower))
    etype = np.mod(rel_adj, _RELNUM)
    M = np.zeros((_RELNUM, _SLEN, _SLEN), np.float32)
    for r in range(_RELNUM):
        mask = (etype == r).astype(np.float32)
        cnt = mask.sum(axis=0)
        Mr = np.where(cnt[None, :] > 0, mask / np.maximum(cnt[None, :], 1.0), 0.0)
        M[r] = Mr.T
    return M


_M_CONST = _build_M()


_IBLK = 8


def _tc_body(comp_ref, x_ref, pad_ref, M_ref, basis_ref, root_ref, bias_ref,
             Wa_ref, ba_ref, Wb_ref, bb_ref, out_ref, n_ref):
    @pl.when(pl.program_id(0) == 0)
    def _():
        for b in range(_NBASES):
            Nb = comp_ref[0, b] * M_ref[0]
            for r in range(1, _RELNUM):
                Nb = Nb + comp_ref[r, b] * M_ref[r]
            n_ref[b] = Nb

    x = x_ref[0]
    pad = pad_ref[0]
    acc = jnp.dot(x, root_ref[...],
                  preferred_element_type=jnp.float32) + bias_ref[...]
    for b in range(_NBASES):
        acc = acc + jnp.dot(
            jnp.dot(n_ref[b], x, preferred_element_type=jnp.float32),
            basis_ref[b], preferred_element_type=jnp.float32)
    ma = jnp.dot(acc, Wa_ref[...],
                 preferred_element_type=jnp.float32) + ba_ref[...]
    mb = jnp.dot(pad, Wb_ref[...],
                 preferred_element_type=jnp.float32) + bb_ref[...]
    S = lax.dot_general(ma, mb, (((1,), (1,)), ((), ())),
                        preferred_element_type=jnp.float32)
    colmax = jnp.max(S, axis=0, keepdims=True)
    E = jnp.exp(S - colmax)
    E = E / jnp.sum(E, axis=0, keepdims=True)
    out_ref[0] = jnp.dot(E, pad, preferred_element_type=jnp.float32)


def _dense_final(comp, x, padded, M, basis, root, bias2, Wa, ba2, Wb, bb2):
    full = lambda shape: pl.BlockSpec(shape, lambda b: (0,) * len(shape))
    return pl.pallas_call(
        _tc_body,
        grid=(_B,),
        in_specs=[
            pl.BlockSpec(memory_space=pltpu.SMEM),
            pl.BlockSpec((1, _SLEN, _D), lambda b: (b, 0, 0)),
            pl.BlockSpec((1, _SLEN, _D), lambda b: (b, 0, 0)),
            full((_RELNUM, _SLEN, _SLEN)),
            full((_NBASES, _D, _D)),
            full((_D, _D)),
            full((1, _D)),
            full((_D, 600)),
            full((1, 600)),
            full((_D, 600)),
            full((1, 600)),
        ],
        out_specs=pl.BlockSpec((1, _SLEN, _D), lambda b: (b, 0, 0)),
        out_shape=jax.ShapeDtypeStruct((_B, _SLEN, _D), jnp.float32),
        scratch_shapes=[pltpu.VMEM((_NBASES, _SLEN, _SLEN), jnp.float32)],
    )(comp, x, padded, M, basis, root, bias2, Wa, ba2, Wb, bb2)


def _bc_body(xk_ref, xv_ref, outk_ref, outv_ref):
    for x_ref, out_ref in ((xk_ref, outk_ref), (xv_ref, outv_ref)):
        for t in range(_IBLK):
            w = x_ref[t]
            lo = w[:_POSI]
            hi = w[_POSI:, :_SLEN - 128]
            full = jnp.concatenate([lo, hi], axis=1)
            for b in range(_B):
                out_ref[b, t] = full


def _broadcast_windows(x3k, x3v):
    inspec = pl.BlockSpec((_IBLK, 2 * _POSI, 128), lambda i: (i, 0, 0))
    outspec = pl.BlockSpec((_B, _IBLK, _POSI, _SLEN), lambda i: (0, i, 0, 0))
    oshape = jax.ShapeDtypeStruct((_B, _SLEN, _POSI, _SLEN), jnp.float32)
    return pl.pallas_call(
        _bc_body,
        grid=(_SLEN // _IBLK,),
        in_specs=[inspec, inspec],
        out_specs=(outspec, outspec),
        out_shape=(oshape, oshape),
    )(x3k, x3v)


_NROWS = _B * _SLEN
_ROWW = _SLEN * _POSI
_TBL = 400 * _POSI
_NWORK = 32


@functools.cache
def _make_sc_rel_emb():
    @functools.partial(
        pl.kernel,
        out_type=jax.ShapeDtypeStruct((_SLEN * _POSI * 256,), jnp.float32),
        scratch_types=[pltpu.VMEM((_POSI * 512,), jnp.float32),
                       pltpu.VMEM((_POSI * 256,), jnp.float32)],
        mesh=plsc.VectorSubcoreMesh(core_axis_name="c", subcore_axis_name="s"),
        compiler_params=pltpu.CompilerParams(needs_layout_passes=False),
    )
    def _sc_rel_emb(qt_hbm, out_hbm, qt_v, win_v):
        wid = lax.axis_index("s") * 2 + lax.axis_index("c")
        pltpu.sync_copy(qt_hbm, qt_v)
        lanes = lax.iota(jnp.int32, 16)

        def job_body(t, carry):
            i = wid + t * _NWORK

            @pl.when(i < _SLEN)
            def _():
                tbase = 199 - i

                def p_body(p, c):
                    rowbase = tbase + p * 512
                    lo = pl.multiple_of(p * 128, 8)
                    hi = pl.multiple_of(_POSI * 128 + p * 128, 8)
                    for j0 in range(0, 128, 16):
                        vals = plsc.load_gather(qt_v, [rowbase + j0 + lanes])
                        win_v[pl.ds(lo + j0, 16)] = vals
                    for j0 in range(128, 256, 16):
                        vals = plsc.load_gather(qt_v, [rowbase + j0 + lanes])
                        win_v[pl.ds(hi + (j0 - 128), 16)] = vals
                    return c

                lax.fori_loop(0, _POSI, p_body, 0)

                dst = pl.multiple_of(i * (_POSI * 256), 8)
                pltpu.sync_copy(win_v.at[pl.ds(0, _POSI * 256)],
                                out_hbm.at[pl.ds(dst, _POSI * 256)])

            return carry

        lax.fori_loop(0, 7, job_body, 0)

    return _sc_rel_emb


def kernel(x, adj_index, emo_emb, pe_k, pe_v, comp, basis, root, bias,
           Wa, ba, Wb, bb):
    del adj_index
    padded = jnp.concatenate(
        [emo_emb, jnp.zeros((_B, _SLEN, _D - _EMO), jnp.float32)], axis=-1)
    M = jnp.asarray(_M_CONST)
    ztail = jnp.zeros((_POSI, 512 - (_MAXLEN + 1)), jnp.float32)
    qtk = jnp.concatenate([pe_k[::-1].T, ztail], axis=1).reshape(-1)
    qtv = jnp.concatenate([pe_v[::-1].T, ztail], axis=1).reshape(-1)
    sc = _make_sc_rel_emb()
    outk = sc(qtk).reshape(_SLEN, 2 * _POSI, 128)
    outv = sc(qtv).reshape(_SLEN, 2 * _POSI, 128)
    final = _dense_final(comp, x, padded, M, basis, root,
                         bias.reshape(1, _D), Wa, ba.reshape(1, 600),
                         Wb, bb.reshape(1, 600))
    bk, bv = _broadcast_windows(outk, outv)
    rel_emb_k = jnp.swapaxes(bk, 2, 3)
    rel_emb_v = jnp.swapaxes(bv, 2, 3)
    return final, rel_emb_k, rel_emb_v


# speedup vs baseline: 1.1809x; 1.0366x over previous
import functools

import numpy as np
import jax
import jax.numpy as jnp
from jax import lax
from jax.experimental import pallas as pl
from jax.experimental.pallas import tpu as pltpu
from jax.experimental.pallas import tpu_sc as plsc

_B, _SLEN, _D = 8, 200, 300
_MAXLEN, _POSI, _WINDOW, _NBASES = 200, 32, 10, 4
_RELNUM = _WINDOW + 2
_EMO = 200


def _build_M() -> np.ndarray:
    i = np.arange(_SLEN)[:, None]
    j = np.arange(_SLEN)[None, :]
    d = i - j
    lower = -np.minimum((d + 1) // 2, _WINDOW + 1)
    rel_adj = np.where(j > i, 1, np.where(j == i, 0, lower))
    etype = np.mod(rel_adj, _RELNUM)
    M = np.zeros((_RELNUM, _SLEN, _SLEN), np.float32)
    for r in range(_RELNUM):
        mask = (etype == r).astype(np.float32)
        cnt = mask.sum(axis=0)
        Mr = np.where(cnt[None, :] > 0, mask / np.maximum(cnt[None, :], 1.0), 0.0)
        M[r] = Mr.T
    return M


_M_CONST = _build_M()


_IBLK = 25


def _tc_body(comp_ref, x_ref, pad_ref, M_ref, basis_ref, root_ref, bias_ref,
             Wa_ref, ba_ref, Wb_ref, bb_ref, out_ref, n_ref):
    @pl.when(pl.program_id(0) == 0)
    def _():
        for b in range(_NBASES):
            Nb = comp_ref[0, b] * M_ref[0]
            for r in range(1, _RELNUM):
                Nb = Nb + comp_ref[r, b] * M_ref[r]
            n_ref[b] = Nb

    x = x_ref[0]
    pad = pad_ref[0]
    acc = jnp.dot(x, root_ref[...],
                  preferred_element_type=jnp.float32) + bias_ref[...]
    for b in range(_NBASES):
        acc = acc + jnp.dot(
            jnp.dot(n_ref[b], x, preferred_element_type=jnp.float32),
            basis_ref[b], preferred_element_type=jnp.float32)
    ma = jnp.dot(acc, Wa_ref[...],
                 preferred_element_type=jnp.float32) + ba_ref[...]
    mb = jnp.dot(pad, Wb_ref[...],
                 preferred_element_type=jnp.float32) + bb_ref[...]
    S = lax.dot_general(ma, mb, (((1,), (1,)), ((), ())),
                        preferred_element_type=jnp.float32)
    colmax = jnp.max(S, axis=0, keepdims=True)
    E = jnp.exp(S - colmax)
    E = E / jnp.sum(E, axis=0, keepdims=True)
    out_ref[0] = jnp.dot(E, pad, preferred_element_type=jnp.float32)


def _dense_final(comp, x, padded, M, basis, root, bias2, Wa, ba2, Wb, bb2):
    full = lambda shape: pl.BlockSpec(shape, lambda b: (0,) * len(shape))
    return pl.pallas_call(
        _tc_body,
        grid=(_B,),
        in_specs=[
            pl.BlockSpec(memory_space=pltpu.SMEM),
            pl.BlockSpec((1, _SLEN, _D), lambda b: (b, 0, 0)),
            pl.BlockSpec((1, _SLEN, _D), lambda b: (b, 0, 0)),
            full((_RELNUM, _SLEN, _SLEN)),
            full((_NBASES, _D, _D)),
            full((_D, _D)),
            full((1, _D)),
            full((_D, 600)),
            full((1, 600)),
            full((_D, 600)),
            full((1, 600)),
        ],
        out_specs=pl.BlockSpec((1, _SLEN, _D), lambda b: (b, 0, 0)),
        out_shape=jax.ShapeDtypeStruct((_B, _SLEN, _D), jnp.float32),
        scratch_shapes=[pltpu.VMEM((_NBASES, _SLEN, _SLEN), jnp.float32)],
    )(comp, x, padded, M, basis, root, bias2, Wa, ba2, Wb, bb2)


def _bc_body(xk_ref, xv_ref, outk_ref, outv_ref):
    for x_ref, out_ref in ((xk_ref, outk_ref), (xv_ref, outv_ref)):
        for t in range(_IBLK):
            w = x_ref[t]
            lo = w[:_POSI]
            hi = w[_POSI:, :_SLEN - 128]
            full = jnp.concatenate([lo, hi], axis=1)
            for b in range(_B):
                out_ref[b, t] = full


def _broadcast_windows(x3k, x3v):
    inspec = pl.BlockSpec((_IBLK, 2 * _POSI, 128), lambda i: (i, 0, 0))
    outspec = pl.BlockSpec((_B, _IBLK, _POSI, _SLEN), lambda i: (0, i, 0, 0))
    oshape = jax.ShapeDtypeStruct((_B, _SLEN, _POSI, _SLEN), jnp.float32)
    return pl.pallas_call(
        _bc_body,
        grid=(_SLEN // _IBLK,),
        in_specs=[inspec, inspec],
        out_specs=(outspec, outspec),
        out_shape=(oshape, oshape),
    )(x3k, x3v)


_NROWS = _B * _SLEN
_ROWW = _SLEN * _POSI
_TBL = 400 * _POSI
_NWORK = 32


@functools.cache
def _make_sc_rel_emb():
    @functools.partial(
        pl.kernel,
        out_type=jax.ShapeDtypeStruct((_SLEN * _POSI * 256,), jnp.float32),
        scratch_types=[pltpu.VMEM((_POSI * 512,), jnp.float32),
                       pltpu.VMEM((_POSI * 256,), jnp.float32)],
        mesh=plsc.VectorSubcoreMesh(core_axis_name="c", subcore_axis_name="s"),
        compiler_params=pltpu.CompilerParams(needs_layout_passes=False),
    )
    def _sc_rel_emb(qt_hbm, out_hbm, qt_v, win_v):
        wid = lax.axis_index("s") * 2 + lax.axis_index("c")
        pltpu.sync_copy(qt_hbm, qt_v)
        lanes = lax.iota(jnp.int32, 16)

        def job_body(t, carry):
            i = wid + t * _NWORK

            @pl.when(i < _SLEN)
            def _():
                tbase = 199 - i

                def p_body(p, c):
                    rowbase = tbase + p * 512
                    lo = pl.multiple_of(p * 128, 8)
                    hi = pl.multiple_of(_POSI * 128 + p * 128, 8)
                    for j0 in range(0, 128, 16):
                        vals = plsc.load_gather(qt_v, [rowbase + j0 + lanes])
                        win_v[pl.ds(lo + j0, 16)] = vals
                    for j0 in range(128, 256, 16):
                        vals = plsc.load_gather(qt_v, [rowbase + j0 + lanes])
                        win_v[pl.ds(hi + (j0 - 128), 16)] = vals
                    return c

                lax.fori_loop(0, _POSI, p_body, 0)

                dst = pl.multiple_of(i * (_POSI * 256), 8)
                pltpu.sync_copy(win_v.at[pl.ds(0, _POSI * 256)],
                                out_hbm.at[pl.ds(dst, _POSI * 256)])

            return carry

        lax.fori_loop(0, 7, job_body, 0)

    return _sc_rel_emb


def kernel(x, adj_index, emo_emb, pe_k, pe_v, comp, basis, root, bias,
           Wa, ba, Wb, bb):
    del adj_index
    padded = jnp.concatenate(
        [emo_emb, jnp.zeros((_B, _SLEN, _D - _EMO), jnp.float32)], axis=-1)
    M = jnp.asarray(_M_CONST)
    ztail = jnp.zeros((_POSI, 512 - (_MAXLEN + 1)), jnp.float32)
    qtk = jnp.concatenate([pe_k[::-1].T, ztail], axis=1).reshape(-1)
    qtv = jnp.concatenate([pe_v[::-1].T, ztail], axis=1).reshape(-1)
    sc = _make_sc_rel_emb()
    outk = sc(qtk).reshape(_SLEN, 2 * _POSI, 128)
    outv = sc(qtv).reshape(_SLEN, 2 * _POSI, 128)
    final = _dense_final(comp, x, padded, M, basis, root,
                         bias.reshape(1, _D), Wa, ba.reshape(1, 600),
                         Wb, bb.reshape(1, 600))
    bk, bv = _broadcast_windows(outk, outv)
    rel_emb_k = jnp.swapaxes(bk, 2, 3)
    rel_emb_v = jnp.swapaxes(bv, 2, 3)
    return final, rel_emb_k, rel_emb_v


# speedup vs baseline: 1.2526x; 1.0607x over previous
import functools

import numpy as np
import jax
import jax.numpy as jnp
from jax import lax
from jax.experimental import pallas as pl
from jax.experimental.pallas import tpu as pltpu
from jax.experimental.pallas import tpu_sc as plsc

_B, _SLEN, _D = 8, 200, 300
_MAXLEN, _POSI, _WINDOW, _NBASES = 200, 32, 10, 4
_RELNUM = _WINDOW + 2
_EMO = 200


def _build_M() -> np.ndarray:
    i = np.arange(_SLEN)[:, None]
    j = np.arange(_SLEN)[None, :]
    d = i - j
    lower = -np.minimum((d + 1) // 2, _WINDOW + 1)
    rel_adj = np.where(j > i, 1, np.where(j == i, 0, lower))
    etype = np.mod(rel_adj, _RELNUM)
    M = np.zeros((_RELNUM, _SLEN, _SLEN), np.float32)
    for r in range(_RELNUM):
        mask = (etype == r).astype(np.float32)
        cnt = mask.sum(axis=0)
        Mr = np.where(cnt[None, :] > 0, mask / np.maximum(cnt[None, :], 1.0), 0.0)
        M[r] = Mr.T
    return M


_M_CONST = _build_M()


_IBLK = 25


def _tc_body(comp_ref, x_ref, pad_ref, M_ref, basis_ref, root_ref, bias_ref,
             Wa_ref, ba_ref, Wb_ref, bb_ref, out_ref, n_ref):
    @pl.when(pl.program_id(0) == 0)
    def _():
        for b in range(_NBASES):
            Nb = comp_ref[0, b] * M_ref[0]
            for r in range(1, _RELNUM):
                Nb = Nb + comp_ref[r, b] * M_ref[r]
            n_ref[b] = Nb

    x = x_ref[0]
    pad = pad_ref[0]
    acc = jnp.dot(x, root_ref[...],
                  preferred_element_type=jnp.float32) + bias_ref[...]
    for b in range(_NBASES):
        acc = acc + jnp.dot(
            jnp.dot(n_ref[b], x, preferred_element_type=jnp.float32),
            basis_ref[b], preferred_element_type=jnp.float32)
    ma = jnp.dot(acc, Wa_ref[...],
                 preferred_element_type=jnp.float32) + ba_ref[...]
    mb = jnp.dot(pad, Wb_ref[...],
                 preferred_element_type=jnp.float32) + bb_ref[...]
    S = lax.dot_general(ma, mb, (((1,), (1,)), ((), ())),
                        preferred_element_type=jnp.float32)
    colmax = jnp.max(S, axis=0, keepdims=True)
    E = jnp.exp(S - colmax)
    E = E / jnp.sum(E, axis=0, keepdims=True)
    out_ref[0] = jnp.dot(E, pad, preferred_element_type=jnp.float32)


def _dense_final(comp, x, padded, M, basis, root, bias2, Wa, ba2, Wb, bb2):
    full = lambda shape: pl.BlockSpec(shape, lambda b: (0,) * len(shape))
    return pl.pallas_call(
        _tc_body,
        grid=(_B,),
        in_specs=[
            pl.BlockSpec(memory_space=pltpu.SMEM),
            pl.BlockSpec((1, _SLEN, _D), lambda b: (b, 0, 0)),
            pl.BlockSpec((1, _SLEN, _D), lambda b: (b, 0, 0)),
            full((_RELNUM, _SLEN, _SLEN)),
            full((_NBASES, _D, _D)),
            full((_D, _D)),
            full((1, _D)),
            full((_D, 600)),
            full((1, 600)),
            full((_D, 600)),
            full((1, 600)),
        ],
        out_specs=pl.BlockSpec((1, _SLEN, _D), lambda b: (b, 0, 0)),
        out_shape=jax.ShapeDtypeStruct((_B, _SLEN, _D), jnp.float32),
        scratch_shapes=[pltpu.VMEM((_NBASES, _SLEN, _SLEN), jnp.float32)],
    )(comp, x, padded, M, basis, root, bias2, Wa, ba2, Wb, bb2)


def _bc_body(xk_ref, xv_ref, outk_ref, outv_ref):
    for x_ref, out_ref in ((xk_ref, outk_ref), (xv_ref, outv_ref)):
        for t in range(_IBLK):
            w = x_ref[t]
            lo = w[:_POSI]
            hi = w[_POSI:, :_SLEN - 128]
            full = jnp.concatenate([lo, hi], axis=1)
            for b in range(_B):
                out_ref[b, t] = full


def _broadcast_windows(x3k, x3v):
    inspec = pl.BlockSpec((_IBLK, 2 * _POSI, 128), lambda i: (i, 0, 0))
    outspec = pl.BlockSpec((_B, _IBLK, _POSI, _SLEN), lambda i: (0, i, 0, 0))
    oshape = jax.ShapeDtypeStruct((_B, _SLEN, _POSI, _SLEN), jnp.float32)
    return pl.pallas_call(
        _bc_body,
        grid=(_SLEN // _IBLK,),
        in_specs=[inspec, inspec],
        out_specs=(outspec, outspec),
        out_shape=(oshape, oshape),
    )(x3k, x3v)


_NROWS = _B * _SLEN
_ROWW = _SLEN * _POSI
_TBL = 400 * _POSI
_NWORK = 32


@functools.cache
def _make_sc_rel_emb():
    @functools.partial(
        pl.kernel,
        out_type=(jax.ShapeDtypeStruct((_SLEN * _POSI * 256,), jnp.float32),
                  jax.ShapeDtypeStruct((_SLEN * _POSI * 256,), jnp.float32)),
        scratch_types=[pltpu.VMEM((2 * _POSI * 512,), jnp.float32),
                       pltpu.VMEM((_POSI * 256,), jnp.float32)],
        mesh=plsc.VectorSubcoreMesh(core_axis_name="c", subcore_axis_name="s"),
        compiler_params=pltpu.CompilerParams(needs_layout_passes=False),
    )
    def _sc_rel_emb(qt_hbm, outk_hbm, outv_hbm, qt_v, win_v):
        wid = lax.axis_index("s") * 2 + lax.axis_index("c")
        pltpu.sync_copy(qt_hbm, qt_v)
        lanes = lax.iota(jnp.int32, 16)

        def job_body(t, carry):
            job = wid + t * _NWORK

            @pl.when(job < 2 * _SLEN)
            def _():
                tensor = lax.div(job, _SLEN)
                i = job - tensor * _SLEN
                tbase = tensor * (_POSI * 512) + 199 - i

                def p_body(p, c):
                    rowbase = tbase + p * 512
                    lo = pl.multiple_of(p * 128, 8)
                    hi = pl.multiple_of(_POSI * 128 + p * 128, 8)
                    for j0 in range(0, 128, 16):
                        vals = plsc.load_gather(qt_v, [rowbase + j0 + lanes])
                        win_v[pl.ds(lo + j0, 16)] = vals
                    for j0 in range(128, 256, 16):
                        vals = plsc.load_gather(qt_v, [rowbase + j0 + lanes])
                        win_v[pl.ds(hi + (j0 - 128), 16)] = vals
                    return c

                lax.fori_loop(0, _POSI, p_body, 0)

                dst = pl.multiple_of(i * (_POSI * 256), 8)

                @pl.when(tensor == 0)
                def _():
                    pltpu.sync_copy(win_v.at[pl.ds(0, _POSI * 256)],
                                    outk_hbm.at[pl.ds(dst, _POSI * 256)])

                @pl.when(tensor == 1)
                def _():
                    pltpu.sync_copy(win_v.at[pl.ds(0, _POSI * 256)],
                                    outv_hbm.at[pl.ds(dst, _POSI * 256)])

            return carry

        lax.fori_loop(0, 13, job_body, 0)

    return _sc_rel_emb


def kernel(x, adj_index, emo_emb, pe_k, pe_v, comp, basis, root, bias,
           Wa, ba, Wb, bb):
    del adj_index
    padded = jnp.concatenate(
        [emo_emb, jnp.zeros((_B, _SLEN, _D - _EMO), jnp.float32)], axis=-1)
    M = jnp.asarray(_M_CONST)
    ztail = jnp.zeros((_POSI, 512 - (_MAXLEN + 1)), jnp.float32)
    qt = jnp.concatenate([
        jnp.concatenate([pe_k[::-1].T, ztail], axis=1).reshape(-1),
        jnp.concatenate([pe_v[::-1].T, ztail], axis=1).reshape(-1)])
    outk, outv = _make_sc_rel_emb()(qt)
    outk = outk.reshape(_SLEN, 2 * _POSI, 128)
    outv = outv.reshape(_SLEN, 2 * _POSI, 128)
    final = _dense_final(comp, x, padded, M, basis, root,
                         bias.reshape(1, _D), Wa, ba.reshape(1, 600),
                         Wb, bb.reshape(1, 600))
    bk, bv = _broadcast_windows(outk, outv)
    rel_emb_k = jnp.swapaxes(bk, 2, 3)
    rel_emb_v = jnp.swapaxes(bv, 2, 3)
    return final, rel_emb_k, rel_emb_v


# speedup vs baseline: 1.2627x; 1.0080x over previous
import functools

import numpy as np
import jax
import jax.numpy as jnp
from jax import lax
from jax.experimental import pallas as pl
from jax.experimental.pallas import tpu as pltpu
from jax.experimental.pallas import tpu_sc as plsc

_B, _SLEN, _D = 8, 200, 300
_MAXLEN, _POSI, _WINDOW, _NBASES = 200, 32, 10, 4
_RELNUM = _WINDOW + 2
_EMO = 200


def _build_M() -> np.ndarray:
    i = np.arange(_SLEN)[:, None]
    j = np.arange(_SLEN)[None, :]
    d = i - j
    lower = -np.minimum((d + 1) // 2, _WINDOW + 1)
    rel_adj = np.where(j > i, 1, np.where(j == i, 0, lower))
    etype = np.mod(rel_adj, _RELNUM)
    M = np.zeros((_RELNUM, _SLEN, _SLEN), np.float32)
    for r in range(_RELNUM):
        mask = (etype == r).astype(np.float32)
        cnt = mask.sum(axis=0)
        Mr = np.where(cnt[None, :] > 0, mask / np.maximum(cnt[None, :], 1.0), 0.0)
        M[r] = Mr.T
    return M


_M_CONST = _build_M()


_IBLK = 40


def _tc_body(comp_ref, x_ref, pad_ref, M_ref, basis_ref, root_ref, bias_ref,
             Wa_ref, ba_ref, Wb_ref, bb_ref, out_ref, n_ref):
    @pl.when(pl.program_id(0) == 0)
    def _():
        for b in range(_NBASES):
            Nb = comp_ref[0, b] * M_ref[0]
            for r in range(1, _RELNUM):
                Nb = Nb + comp_ref[r, b] * M_ref[r]
            n_ref[b] = Nb

    x = x_ref[0]
    pad = pad_ref[0]
    acc = jnp.dot(x, root_ref[...],
                  preferred_element_type=jnp.float32) + bias_ref[...]
    for b in range(_NBASES):
        acc = acc + jnp.dot(
            jnp.dot(n_ref[b], x, preferred_element_type=jnp.float32),
            basis_ref[b], preferred_element_type=jnp.float32)
    ma = jnp.dot(acc, Wa_ref[...],
                 preferred_element_type=jnp.float32) + ba_ref[...]
    mb = jnp.dot(pad, Wb_ref[...],
                 preferred_element_type=jnp.float32) + bb_ref[...]
    S = lax.dot_general(ma, mb, (((1,), (1,)), ((), ())),
                        preferred_element_type=jnp.float32)
    colmax = jnp.max(S, axis=0, keepdims=True)
    E = jnp.exp(S - colmax)
    E = E / jnp.sum(E, axis=0, keepdims=True)
    out_ref[0] = jnp.dot(E, pad, preferred_element_type=jnp.float32)


def _dense_final(comp, x, padded, M, basis, root, bias2, Wa, ba2, Wb, bb2):
    full = lambda shape: pl.BlockSpec(shape, lambda b: (0,) * len(shape))
    return pl.pallas_call(
        _tc_body,
        grid=(_B,),
        in_specs=[
            pl.BlockSpec(memory_space=pltpu.SMEM),
            pl.BlockSpec((1, _SLEN, _D), lambda b: (b, 0, 0)),
            pl.BlockSpec((1, _SLEN, _D), lambda b: (b, 0, 0)),
            full((_RELNUM, _SLEN, _SLEN)),
            full((_NBASES, _D, _D)),
            full((_D, _D)),
            full((1, _D)),
            full((_D, 600)),
            full((1, 600)),
            full((_D, 600)),
            full((1, 600)),
        ],
        out_specs=pl.BlockSpec((1, _SLEN, _D), lambda b: (b, 0, 0)),
        out_shape=jax.ShapeDtypeStruct((_B, _SLEN, _D), jnp.float32),
        scratch_shapes=[pltpu.VMEM((_NBASES, _SLEN, _SLEN), jnp.float32)],
    )(comp, x, padded, M, basis, root, bias2, Wa, ba2, Wb, bb2)


def _bc_body(xk_ref, xv_ref, outk_ref, outv_ref):
    for x_ref, out_ref in ((xk_ref, outk_ref), (xv_ref, outv_ref)):
        for t in range(_IBLK):
            w = x_ref[t]
            lo = w[:_POSI]
            hi = w[_POSI:, :_SLEN - 128]
            full = jnp.concatenate([lo, hi], axis=1)
            for b in range(_B):
                out_ref[b, t] = full


def _broadcast_windows(x3k, x3v):
    inspec = pl.BlockSpec((_IBLK, 2 * _POSI, 128), lambda i: (i, 0, 0))
    outspec = pl.BlockSpec((_B, _IBLK, _POSI, _SLEN), lambda i: (0, i, 0, 0))
    oshape = jax.ShapeDtypeStruct((_B, _SLEN, _POSI, _SLEN), jnp.float32)
    return pl.pallas_call(
        _bc_body,
        grid=(_SLEN // _IBLK,),
        in_specs=[inspec, inspec],
        out_specs=(outspec, outspec),
        out_shape=(oshape, oshape),
    )(x3k, x3v)


_NROWS = _B * _SLEN
_ROWW = _SLEN * _POSI
_TBL = 400 * _POSI
_NWORK = 32


@functools.cache
def _make_sc_rel_emb():
    @functools.partial(
        pl.kernel,
        out_type=(jax.ShapeDtypeStruct((_SLEN * _POSI * 256,), jnp.float32),
                  jax.ShapeDtypeStruct((_SLEN * _POSI * 256,), jnp.float32)),
        scratch_types=[pltpu.VMEM((2 * _POSI * 512,), jnp.float32),
                       pltpu.VMEM((_POSI * 256,), jnp.float32)],
        mesh=plsc.VectorSubcoreMesh(core_axis_name="c", subcore_axis_name="s"),
        compiler_params=pltpu.CompilerParams(needs_layout_passes=False),
    )
    def _sc_rel_emb(qt_hbm, outk_hbm, outv_hbm, qt_v, win_v):
        wid = lax.axis_index("s") * 2 + lax.axis_index("c")
        pltpu.sync_copy(qt_hbm, qt_v)
        lanes = lax.iota(jnp.int32, 16)

        def job_body(t, carry):
            job = wid + t * _NWORK

            @pl.when(job < 2 * _SLEN)
            def _():
                tensor = lax.div(job, _SLEN)
                i = job - tensor * _SLEN
                tbase = tensor * (_POSI * 512) + 199 - i

                def p_body(p, c):
                    rowbase = tbase + p * 512
                    lo = pl.multiple_of(p * 128, 8)
                    hi = pl.multiple_of(_POSI * 128 + p * 128, 8)
                    for j0 in range(0, 128, 16):
                        vals = plsc.load_gather(qt_v, [rowbase + j0 + lanes])
                        win_v[pl.ds(lo + j0, 16)] = vals
                    for j0 in range(128, 256, 16):
                        vals = plsc.load_gather(qt_v, [rowbase + j0 + lanes])
                        win_v[pl.ds(hi + (j0 - 128), 16)] = vals
                    return c

                lax.fori_loop(0, _POSI, p_body, 0)

                dst = pl.multiple_of(i * (_POSI * 256), 8)

                @pl.when(tensor == 0)
                def _():
                    pltpu.sync_copy(win_v.at[pl.ds(0, _POSI * 256)],
                                    outk_hbm.at[pl.ds(dst, _POSI * 256)])

                @pl.when(tensor == 1)
                def _():
                    pltpu.sync_copy(win_v.at[pl.ds(0, _POSI * 256)],
                                    outv_hbm.at[pl.ds(dst, _POSI * 256)])

            return carry

        lax.fori_loop(0, 13, job_body, 0)

    return _sc_rel_emb


def kernel(x, adj_index, emo_emb, pe_k, pe_v, comp, basis, root, bias,
           Wa, ba, Wb, bb):
    del adj_index
    padded = jnp.concatenate(
        [emo_emb, jnp.zeros((_B, _SLEN, _D - _EMO), jnp.float32)], axis=-1)
    M = jnp.asarray(_M_CONST)
    ztail = jnp.zeros((_POSI, 512 - (_MAXLEN + 1)), jnp.float32)
    qt = jnp.concatenate([
        jnp.concatenate([pe_k[::-1].T, ztail], axis=1).reshape(-1),
        jnp.concatenate([pe_v[::-1].T, ztail], axis=1).reshape(-1)])
    outk, outv = _make_sc_rel_emb()(qt)
    outk = outk.reshape(_SLEN, 2 * _POSI, 128)
    outv = outv.reshape(_SLEN, 2 * _POSI, 128)
    final = _dense_final(comp, x, padded, M, basis, root,
                         bias.reshape(1, _D), Wa, ba.reshape(1, 600),
                         Wb, bb.reshape(1, 600))
    bk, bv = _broadcast_windows(outk, outv)
    rel_emb_k = jnp.swapaxes(bk, 2, 3)
    rel_emb_v = jnp.swapaxes(bv, 2, 3)
    return final, rel_emb_k, rel_emb_v
